# baseline (speedup 1.0000x reference)
import functools

import jax
import jax.numpy as jnp
from jax import lax
from jax.experimental import pallas as pl
from jax.experimental.pallas import tpu as pltpu
from jax.experimental.pallas import tpu_sc as plsc

N = 10000
D = 128
E = 320000
NC = 2
NS = 16
NW = NC * NS
CB = 128
SUPC = 8
NSUP = 10
CH = NSUP * SUPC
EPAD = NW * CB * CH
NPAD = 10240
RPT = NPAD // NS
RCH = RPT // CB
CW = 16
BM = 1024
G = NPAD // BM

_mesh = plsc.VectorSubcoreMesh(core_axis_name="c", subcore_axis_name="s")


def _fill_own_rows(own_v, base):
    def fill_own(tt, carry):
        own_v[tt // 8, pl.ds((tt % 8) * 16, 16)] = (
            base + tt * 16 + lax.iota(jnp.int32, 16))
        return carry

    lax.fori_loop(0, RPT // 16, fill_own, 0)


@functools.partial(
    pl.kernel,
    out_type=jax.ShapeDtypeStruct((NC, NPAD, CW), jnp.float32),
    mesh=_mesh,
    scratch_types=[
        pltpu.VMEM((CB, CW), jnp.float32),
        pltpu.VMEM((CB, CW), jnp.float32),
        pltpu.VMEM(((NSUP + 1) * SUPC, CB), jnp.int32),
        pltpu.VMEM((RCH, CB), jnp.int32),
        pltpu.VMEM_SHARED((NPAD, CW), jnp.float32),
    ],
)
def _sc_counts(dst_hbm, out, buf_v, ones_v, dst_v, own_v, acc_sh):
    c = lax.axis_index("c")
    s = lax.axis_index("s")
    wid = c * NS + s
    pltpu.sync_copy(dst_hbm.at[wid], dst_v)

    def fill(i, carry):
        buf_v[i, :] = jnp.full((CW,), 0.0, jnp.float32)
        ones_v[i, :] = jnp.full((CW,), 1.0, jnp.float32)
        return carry

    lax.fori_loop(0, CB, fill, 0)
    base = s * RPT
    _fill_own_rows(own_v, base)

    for k in range(RCH):
        pltpu.sync_copy(buf_v, acc_sh.at[own_v.at[k]])
    plsc.subcore_barrier()

    def body(j, carry):
        pltpu.sync_copy(ones_v, acc_sh.at[dst_v.at[j]], add=True)
        return carry

    lax.fori_loop(0, CH, body, 0)
    plsc.subcore_barrier()

    for k in range(RCH):
        pltpu.sync_copy(acc_sh.at[own_v.at[k]], buf_v)
        pltpu.sync_copy(buf_v, out.at[c, pl.ds(base + k * CB, CB)])


@functools.partial(
    pl.kernel,
    out_type=jax.ShapeDtypeStruct((NC, NPAD, D), jnp.float32),
    mesh=_mesh,
    scratch_types=[
        pltpu.VMEM((2, SUPC, CB), jnp.int32),
        pltpu.VMEM((2, SUPC, CB), jnp.int32),
        pltpu.VMEM((CB, D), jnp.float32),
        pltpu.VMEM((CB, D), jnp.float32),
        pltpu.VMEM((RCH, CB), jnp.int32),
        pltpu.VMEM_SHARED((NPAD, D), jnp.float32),
        pltpu.SemaphoreType.DMA,
        pltpu.SemaphoreType.DMA,
        pltpu.SemaphoreType.DMA,
        pltpu.SemaphoreType.DMA,
        pltpu.SemaphoreType.DMA,
        pltpu.SemaphoreType.DMA,
    ],
)
def _sc_scatter(table_hbm, src_hbm, dst_hbm, out, src_s, dst_s, rows_a,
                rows_b, own_v, acc_sh, sem_ga, sem_gb, sem_sa, sem_sb,
                sem_is, sem_id):
    c = lax.axis_index("c")
    s = lax.axis_index("s")
    wid = c * NS + s

    def fill(i, carry):
        for q in range(D // 16):
            rows_a[i, pl.ds(q * 16, 16)] = jnp.full((16,), 0.0, jnp.float32)
        return carry

    lax.fori_loop(0, CB, fill, 0)
    base = s * RPT
    _fill_own_rows(own_v, base)

    for k in range(RCH):
        pltpu.sync_copy(rows_a, acc_sh.at[own_v.at[k]])
    plsc.subcore_barrier()

    bufs = (rows_a, rows_b)
    gsems = (sem_ga, sem_gb)
    ssems = (sem_sa, sem_sb)

    def i_start(sup, slot):
        pltpu.async_copy(src_hbm.at[wid, sup], src_s.at[slot], sem_is)
        pltpu.async_copy(dst_hbm.at[wid, sup], dst_s.at[slot], sem_id)

    def i_wait(sup, slot):
        pltpu.make_async_copy(src_hbm.at[wid, sup], src_s.at[slot], sem_is).wait()
        pltpu.make_async_copy(dst_hbm.at[wid, sup], dst_s.at[slot], sem_id).wait()

    def g_start(slot, q):
        pltpu.async_copy(table_hbm.at[src_s.at[slot, q]], bufs[q % 2],
                         gsems[q % 2])

    def g_wait(slot, q):
        pltpu.make_async_copy(table_hbm.at[src_s.at[slot, q]], bufs[q % 2],
                              gsems[q % 2]).wait()

    def s_start(slot, q):
        pltpu.async_copy(bufs[q % 2], acc_sh.at[dst_s.at[slot, q]],
                         ssems[q % 2], add=True)

    def s_wait(slot, q):
        pltpu.make_async_copy(bufs[q % 2], acc_sh.at[dst_s.at[slot, q]],
                              ssems[q % 2]).wait()

    i_start(0, 0)

    def body(p, carry):
        slot = lax.rem(p, 2)
        i_wait(p, slot)
        i_start(p + 1, 1 - slot)
        g_start(slot, 0)
        for q in range(SUPC):
            g_wait(slot, q)
            if q + 1 < SUPC:
                if q >= 1:
                    s_wait(slot, q - 1)
                g_start(slot, q + 1)
            s_start(slot, q)
        s_wait(slot, SUPC - 1)
        return carry

    lax.fori_loop(0, NSUP, body, 0)
    i_wait(NSUP, lax.rem(jnp.int32(NSUP), 2))
    plsc.subcore_barrier()

    for k in range(RCH):
        pltpu.sync_copy(acc_sh.at[own_v.at[k]], rows_a)
        pltpu.sync_copy(rows_a, out.at[c, pl.ds(base + k * CB, CB)])


def _dis_of(c0_ref, c1_ref):
    deg = 1.0 + c0_ref[0, :, 0:1] + c1_ref[0, :, 0:1]
    return lax.rsqrt(deg)


def _stage_first_body(x_ref, w_ref, c0_ref, c1_ref, o_ref):
    dis = _dis_of(c0_ref, c1_ref)
    o_ref[...] = jnp.dot(x_ref[...], w_ref[...],
                         preferred_element_type=jnp.float32) * dis


def _stage_mid_body(p0_ref, p1_ref, g_ref, c0_ref, c1_ref, b_ref, w_ref, o_ref):
    dis = _dis_of(c0_ref, c1_ref)
    h = jnp.maximum(
        dis * (p0_ref[0] + p1_ref[0] + g_ref[...]) + b_ref[...], 0.0)
    o_ref[...] = jnp.dot(h, w_ref[...], preferred_element_type=jnp.float32) * dis


def _stage_last_body(p0_ref, p1_ref, g_ref, c0_ref, c1_ref, b_ref, wf_ref,
                     bf_ref, o_ref):
    dis = _dis_of(c0_ref, c1_ref)
    h = jnp.maximum(
        dis * (p0_ref[0] + p1_ref[0] + g_ref[...]) + b_ref[...], 0.0)
    o_ref[...] = jnp.dot(h, wf_ref[...],
                         preferred_element_type=jnp.float32) + bf_ref[...]


_row_spec = pl.BlockSpec((BM, D), lambda i: (i, 0))
_prt_spec = pl.BlockSpec((1, BM, D), lambda i: (0, i, 0))
_cnt_spec = pl.BlockSpec((1, BM, CW), lambda i: (0, i, 0))
_w_spec = pl.BlockSpec((D, D), lambda i: (0, 0))
_b_spec = pl.BlockSpec((1, D), lambda i: (0, 0))

_stage_first = pl.pallas_call(
    _stage_first_body,
    grid=(G,),
    in_specs=[_row_spec, _w_spec, _cnt_spec, _cnt_spec],
    out_specs=_row_spec,
    out_shape=jax.ShapeDtypeStruct((NPAD, D), jnp.float32),
)

_stage_mid = pl.pallas_call(
    _stage_mid_body,
    grid=(G,),
    in_specs=[_prt_spec, _prt_spec, _row_spec, _cnt_spec, _cnt_spec, _b_spec,
              _w_spec],
    out_specs=_row_spec,
    out_shape=jax.ShapeDtypeStruct((NPAD, D), jnp.float32),
)

_stage_last = pl.pallas_call(
    _stage_last_body,
    grid=(G,),
    in_specs=[_prt_spec, _prt_spec, _row_spec, _cnt_spec, _cnt_spec, _b_spec,
              pl.BlockSpec((D, 1), lambda i: (0, 0)),
              pl.BlockSpec((1, 1), lambda i: (0, 0))],
    out_specs=pl.BlockSpec((BM, 1), lambda i: (i, 0)),
    out_shape=jax.ShapeDtypeStruct((NPAD, 1), jnp.float32),
)


def kernel(x, edge_index, W1, b1, W2, b2, W3, b3, Wf, bf):
    x_pad = jnp.zeros((NPAD, D), jnp.float32).at[:N].set(x)
    pad = EPAD - E

    def stage(idx):
        main = jnp.concatenate(
            [idx, jnp.full((pad,), N, jnp.int32)]).reshape(NW, NSUP, SUPC, CB)
        extra = jnp.full((NW, 1, SUPC, CB), N, jnp.int32)
        return jnp.concatenate([main, extra], axis=1)

    src_t = stage(edge_index[0])
    dst_t = stage(edge_index[1])
    dst_flat = dst_t.reshape(NW, (NSUP + 1) * SUPC, CB)

    cc = _sc_counts(dst_flat)
    c0, c1 = cc[0:1], cc[1:2]
    b1r = b1.reshape(1, D)
    b2r = b2.reshape(1, D)
    b3r = b3.reshape(1, D)
    bfr = bf.reshape(1, 1)

    g = _stage_first(x_pad, W1, c0, c1)
    pp = _sc_scatter(g, src_t, dst_t)
    g = _stage_mid(pp[0:1], pp[1:2], g, c0, c1, b1r, W2)
    pp = _sc_scatter(g, src_t, dst_t)
    g = _stage_mid(pp[0:1], pp[1:2], g, c0, c1, b2r, W3)
    pp = _sc_scatter(g, src_t, dst_t)
    out = _stage_last(pp[0:1], pp[1:2], g, c0, c1, b3r, Wf, bfr)
    return out[:N]


# speedup vs baseline: 1.0004x; 1.0004x over previous
import functools

import jax
import jax.numpy as jnp
from jax import lax
from jax.experimental import pallas as pl
from jax.experimental.pallas import tpu as pltpu
from jax.experimental.pallas import tpu_sc as plsc

N = 10000
D = 128
E = 320000
NC = 2
NS = 16
NW = NC * NS
CB = 128
SUPC = 8
NSUP = 10
CH = NSUP * SUPC
EPAD = NW * CB * CH
NPAD = 10240
RPT = NPAD // NS
RCH = RPT // CB
CW = 16
BM = 1024
G = NPAD // BM

_mesh = plsc.VectorSubcoreMesh(core_axis_name="c", subcore_axis_name="s")


def _fill_own_rows(own_v, base):
    def fill_own(tt, carry):
        own_v[tt // 8, pl.ds((tt % 8) * 16, 16)] = (
            base + tt * 16 + lax.iota(jnp.int32, 16))
        return carry

    lax.fori_loop(0, RPT // 16, fill_own, 0)


@functools.partial(
    pl.kernel,
    out_type=jax.ShapeDtypeStruct((NC, NPAD, CW), jnp.float32),
    mesh=_mesh,
    scratch_types=[
        pltpu.VMEM((CB, CW), jnp.float32),
        pltpu.VMEM((CB, CW), jnp.float32),
        pltpu.VMEM(((NSUP + 1) * SUPC, CB), jnp.int32),
        pltpu.VMEM((RCH, CB), jnp.int32),
        pltpu.VMEM_SHARED((NPAD, CW), jnp.float32),
    ],
)
def _sc_counts(dst_hbm, out, buf_v, ones_v, dst_v, own_v, acc_sh):
    c = lax.axis_index("c")
    s = lax.axis_index("s")
    wid = c * NS + s
    pltpu.sync_copy(dst_hbm.at[wid], dst_v)

    def fill(i, carry):
        buf_v[i, :] = jnp.full((CW,), 0.0, jnp.float32)
        ones_v[i, :] = jnp.full((CW,), 1.0, jnp.float32)
        return carry

    lax.fori_loop(0, CB, fill, 0)
    base = s * RPT
    _fill_own_rows(own_v, base)

    for k in range(RCH):
        pltpu.sync_copy(buf_v, acc_sh.at[own_v.at[k]])
    plsc.subcore_barrier()

    def body(j, carry):
        pltpu.sync_copy(ones_v, acc_sh.at[dst_v.at[j]], add=True)
        return carry

    lax.fori_loop(0, CH, body, 0)
    plsc.subcore_barrier()

    for k in range(RCH):
        pltpu.sync_copy(acc_sh.at[own_v.at[k]], buf_v)
        pltpu.sync_copy(buf_v, out.at[c, pl.ds(base + k * CB, CB)])


@functools.partial(
    pl.kernel,
    out_type=jax.ShapeDtypeStruct((NC, NPAD, D), jnp.float32),
    mesh=_mesh,
    scratch_types=[
        pltpu.VMEM((2, SUPC, CB), jnp.int32),
        pltpu.VMEM((2, SUPC, CB), jnp.int32),
        pltpu.VMEM((CB, D), jnp.float32),
        pltpu.VMEM((CB, D), jnp.float32),
        pltpu.VMEM((RCH, CB), jnp.int32),
        pltpu.VMEM_SHARED((NPAD, D), jnp.float32),
        pltpu.SemaphoreType.DMA,
        pltpu.SemaphoreType.DMA,
        pltpu.SemaphoreType.DMA,
        pltpu.SemaphoreType.DMA,
        pltpu.SemaphoreType.DMA,
        pltpu.SemaphoreType.DMA,
    ],
)
def _sc_scatter(table_hbm, src_hbm, dst_hbm, out, src_s, dst_s, rows_a,
                rows_b, own_v, acc_sh, sem_ga, sem_gb, sem_sa, sem_sb,
                sem_is, sem_id):
    c = lax.axis_index("c")
    s = lax.axis_index("s")
    wid = c * NS + s

    def fill(i, carry):
        for q in range(D // 16):
            rows_a[i, pl.ds(q * 16, 16)] = jnp.full((16,), 0.0, jnp.float32)
        return carry

    lax.fori_loop(0, CB, fill, 0)
    base = s * RPT
    _fill_own_rows(own_v, base)

    for k in range(RCH):
        pltpu.sync_copy(rows_a, acc_sh.at[own_v.at[k]])
    plsc.subcore_barrier()

    bufs = (rows_a, rows_b)
    gsems = (sem_ga, sem_gb)
    ssems = (sem_sa, sem_sb)

    def i_start(sup, slot):
        pltpu.async_copy(src_hbm.at[wid, sup], src_s.at[slot], sem_is)
        pltpu.async_copy(dst_hbm.at[wid, sup], dst_s.at[slot], sem_id)

    def i_wait(sup, slot):
        pltpu.make_async_copy(src_hbm.at[wid, sup], src_s.at[slot], sem_is).wait()
        pltpu.make_async_copy(dst_hbm.at[wid, sup], dst_s.at[slot], sem_id).wait()

    def g_start(slot, q):
        return pltpu.async_copy(table_hbm.at[src_s.at[slot, q]], bufs[q % 2],
                                gsems[q % 2])

    def s_start(slot, q):
        return pltpu.async_copy(bufs[q % 2], acc_sh.at[dst_s.at[slot, q]],
                                ssems[q % 2], add=True)

    i_start(0, 0)

    def body(p, carry):
        slot = lax.rem(p, 2)
        i_wait(p, slot)
        i_start(p + 1, 1 - slot)
        hg = [None] * SUPC
        hs = [None] * SUPC
        hg[0] = g_start(slot, 0)
        for q in range(SUPC):
            hg[q].wait()
            if q + 1 < SUPC:
                if q >= 1:
                    hs[q - 1].wait()
                hg[q + 1] = g_start(slot, q + 1)
            hs[q] = s_start(slot, q)
        hs[SUPC - 1].wait()
        return carry

    lax.fori_loop(0, NSUP, body, 0)
    i_wait(NSUP, lax.rem(jnp.int32(NSUP), 2))
    plsc.subcore_barrier()

    for k in range(RCH):
        pltpu.sync_copy(acc_sh.at[own_v.at[k]], rows_a)
        pltpu.sync_copy(rows_a, out.at[c, pl.ds(base + k * CB, CB)])


def _dis_of(c0_ref, c1_ref):
    deg = 1.0 + c0_ref[0, :, 0:1] + c1_ref[0, :, 0:1]
    return lax.rsqrt(deg)


def _stage_first_body(x_ref, w_ref, c0_ref, c1_ref, o_ref):
    dis = _dis_of(c0_ref, c1_ref)
    o_ref[...] = jnp.dot(x_ref[...], w_ref[...],
                         preferred_element_type=jnp.float32) * dis


def _stage_mid_body(p0_ref, p1_ref, g_ref, c0_ref, c1_ref, b_ref, w_ref, o_ref):
    dis = _dis_of(c0_ref, c1_ref)
    h = jnp.maximum(
        dis * (p0_ref[0] + p1_ref[0] + g_ref[...]) + b_ref[...], 0.0)
    o_ref[...] = jnp.dot(h, w_ref[...], preferred_element_type=jnp.float32) * dis


def _stage_last_body(p0_ref, p1_ref, g_ref, c0_ref, c1_ref, b_ref, wf_ref,
                     bf_ref, o_ref):
    dis = _dis_of(c0_ref, c1_ref)
    h = jnp.maximum(
        dis * (p0_ref[0] + p1_ref[0] + g_ref[...]) + b_ref[...], 0.0)
    o_ref[...] = jnp.dot(h, wf_ref[...],
                         preferred_element_type=jnp.float32) + bf_ref[...]


_row_spec = pl.BlockSpec((BM, D), lambda i: (i, 0))
_prt_spec = pl.BlockSpec((1, BM, D), lambda i: (0, i, 0))
_cnt_spec = pl.BlockSpec((1, BM, CW), lambda i: (0, i, 0))
_w_spec = pl.BlockSpec((D, D), lambda i: (0, 0))
_b_spec = pl.BlockSpec((1, D), lambda i: (0, 0))

_stage_first = pl.pallas_call(
    _stage_first_body,
    grid=(G,),
    in_specs=[_row_spec, _w_spec, _cnt_spec, _cnt_spec],
    out_specs=_row_spec,
    out_shape=jax.ShapeDtypeStruct((NPAD, D), jnp.float32),
)

_stage_mid = pl.pallas_call(
    _stage_mid_body,
    grid=(G,),
    in_specs=[_prt_spec, _prt_spec, _row_spec, _cnt_spec, _cnt_spec, _b_spec,
              _w_spec],
    out_specs=_row_spec,
    out_shape=jax.ShapeDtypeStruct((NPAD, D), jnp.float32),
)

_stage_last = pl.pallas_call(
    _stage_last_body,
    grid=(G,),
    in_specs=[_prt_spec, _prt_spec, _row_spec, _cnt_spec, _cnt_spec, _b_spec,
              pl.BlockSpec((D, 1), lambda i: (0, 0)),
              pl.BlockSpec((1, 1), lambda i: (0, 0))],
    out_specs=pl.BlockSpec((BM, 1), lambda i: (i, 0)),
    out_shape=jax.ShapeDtypeStruct((NPAD, 1), jnp.float32),
)


def kernel(x, edge_index, W1, b1, W2, b2, W3, b3, Wf, bf):
    x_pad = jnp.zeros((NPAD, D), jnp.float32).at[:N].set(x)
    pad = EPAD - E

    def stage(idx):
        main = jnp.concatenate(
            [idx, jnp.full((pad,), N, jnp.int32)]).reshape(NW, NSUP, SUPC, CB)
        extra = jnp.full((NW, 1, SUPC, CB), N, jnp.int32)
        return jnp.concatenate([main, extra], axis=1)

    src_t = stage(edge_index[0])
    dst_t = stage(edge_index[1])
    dst_flat = dst_t.reshape(NW, (NSUP + 1) * SUPC, CB)

    cc = _sc_counts(dst_flat)
    c0, c1 = cc[0:1], cc[1:2]
    b1r = b1.reshape(1, D)
    b2r = b2.reshape(1, D)
    b3r = b3.reshape(1, D)
    bfr = bf.reshape(1, 1)

    g = _stage_first(x_pad, W1, c0, c1)
    pp = _sc_scatter(g, src_t, dst_t)
    g = _stage_mid(pp[0:1], pp[1:2], g, c0, c1, b1r, W2)
    pp = _sc_scatter(g, src_t, dst_t)
    g = _stage_mid(pp[0:1], pp[1:2], g, c0, c1, b2r, W3)
    pp = _sc_scatter(g, src_t, dst_t)
    out = _stage_last(pp[0:1], pp[1:2], g, c0, c1, b3r, Wf, bfr)
    return out[:N]


# speedup vs baseline: 1.3092x; 1.3087x over previous
import functools

import jax
import jax.numpy as jnp
from jax import lax
from jax.experimental import pallas as pl
from jax.experimental.pallas import tpu as pltpu
from jax.experimental.pallas import tpu_sc as plsc

N = 10000
D = 128
E = 320000
NC = 2
NS = 16
NW = NC * NS
CB = 128
CH = (E + NW * CB - 1) // (NW * CB)
EPAD = NW * CB * CH
NPAD = 10240
RPT = NPAD // NS
RCH = RPT // CB
CW = 16
BM = 1024
G = NPAD // BM

_mesh = plsc.VectorSubcoreMesh(core_axis_name="c", subcore_axis_name="s")


def _fill_own_rows(own_v, base):
    def fill_own(tt, carry):
        own_v[tt // 8, pl.ds((tt % 8) * 16, 16)] = (
            base + tt * 16 + lax.iota(jnp.int32, 16))
        return carry

    lax.fori_loop(0, RPT // 16, fill_own, 0)


@functools.partial(
    pl.kernel,
    out_type=jax.ShapeDtypeStruct((NC, NPAD, CW), jnp.float32),
    mesh=_mesh,
    scratch_types=[
        pltpu.VMEM((CB, CW), jnp.float32),
        pltpu.VMEM((CB, CW), jnp.float32),
        pltpu.VMEM((CH, CB), jnp.int32),
        pltpu.VMEM((RCH, CB), jnp.int32),
        pltpu.VMEM_SHARED((NPAD, CW), jnp.float32),
    ],
)
def _sc_counts(dst_hbm, out, buf_v, ones_v, dst_v, own_v, acc_sh):
    c = lax.axis_index("c")
    s = lax.axis_index("s")
    wid = c * NS + s
    pltpu.sync_copy(dst_hbm.at[wid], dst_v)

    def fill(i, carry):
        buf_v[i, :] = jnp.full((CW,), 0.0, jnp.float32)
        ones_v[i, :] = jnp.full((CW,), 1.0, jnp.float32)
        return carry

    lax.fori_loop(0, CB, fill, 0)
    base = s * RPT
    _fill_own_rows(own_v, base)

    for k in range(RCH):
        pltpu.sync_copy(buf_v, acc_sh.at[own_v.at[k]])
    plsc.subcore_barrier()

    def body(j, carry):
        pltpu.sync_copy(ones_v, acc_sh.at[dst_v.at[j]], add=True)
        return carry

    lax.fori_loop(0, CH, body, 0)
    plsc.subcore_barrier()

    for k in range(RCH):
        pltpu.sync_copy(acc_sh.at[own_v.at[k]], buf_v)
        pltpu.sync_copy(buf_v, out.at[c, pl.ds(base + k * CB, CB)])


@functools.partial(
    pl.kernel,
    out_type=jax.ShapeDtypeStruct((NC, NPAD, D), jnp.float32),
    mesh=_mesh,
    scratch_types=[
        pltpu.VMEM((CH, CB), jnp.int32),
        pltpu.VMEM((CH, CB), jnp.int32),
        pltpu.VMEM((CB, D), jnp.float32),
        pltpu.VMEM((RCH, CB), jnp.int32),
        pltpu.VMEM_SHARED((NPAD, D), jnp.float32),
    ],
)
def _sc_scatter(table_hbm, src_hbm, dst_hbm, out, src_v, dst_v, rows_v,
                own_v, acc_sh):
    c = lax.axis_index("c")
    s = lax.axis_index("s")
    wid = c * NS + s
    pltpu.sync_copy(src_hbm.at[wid], src_v)
    pltpu.sync_copy(dst_hbm.at[wid], dst_v)

    def fill(i, carry):
        for q in range(D // 16):
            rows_v[i, pl.ds(q * 16, 16)] = jnp.full((16,), 0.0, jnp.float32)
        return carry

    lax.fori_loop(0, CB, fill, 0)
    base = s * RPT
    _fill_own_rows(own_v, base)

    for k in range(RCH):
        pltpu.sync_copy(rows_v, acc_sh.at[own_v.at[k]])
    plsc.subcore_barrier()

    def body(j, carry):
        pltpu.sync_copy(table_hbm.at[src_v.at[j]], rows_v)
        pltpu.sync_copy(rows_v, acc_sh.at[dst_v.at[j]], add=True)
        return carry

    lax.fori_loop(0, CH, body, 0)
    plsc.subcore_barrier()

    for k in range(RCH):
        pltpu.sync_copy(acc_sh.at[own_v.at[k]], rows_v)
        pltpu.sync_copy(rows_v, out.at[c, pl.ds(base + k * CB, CB)])


def _dis_of(c0_ref, c1_ref):
    deg = 1.0 + c0_ref[0, :, 0:1] + c1_ref[0, :, 0:1]
    return lax.rsqrt(deg)


def _stage_first_body(x_ref, w_ref, c0_ref, c1_ref, o_ref):
    dis = _dis_of(c0_ref, c1_ref)
    o_ref[...] = jnp.dot(x_ref[...], w_ref[...],
                         preferred_element_type=jnp.float32) * dis


def _stage_mid_body(p0_ref, p1_ref, g_ref, c0_ref, c1_ref, b_ref, w_ref, o_ref):
    dis = _dis_of(c0_ref, c1_ref)
    h = jnp.maximum(
        dis * (p0_ref[0] + p1_ref[0] + g_ref[...]) + b_ref[...], 0.0)
    o_ref[...] = jnp.dot(h, w_ref[...], preferred_element_type=jnp.float32) * dis


def _stage_last_body(p0_ref, p1_ref, g_ref, c0_ref, c1_ref, b_ref, wf_ref,
                     bf_ref, o_ref):
    dis = _dis_of(c0_ref, c1_ref)
    h = jnp.maximum(
        dis * (p0_ref[0] + p1_ref[0] + g_ref[...]) + b_ref[...], 0.0)
    o_ref[...] = jnp.dot(h, wf_ref[...],
                         preferred_element_type=jnp.float32) + bf_ref[...]


_row_spec = pl.BlockSpec((BM, D), lambda i: (i, 0))
_prt_spec = pl.BlockSpec((1, BM, D), lambda i: (0, i, 0))
_cnt_spec = pl.BlockSpec((1, BM, CW), lambda i: (0, i, 0))
_w_spec = pl.BlockSpec((D, D), lambda i: (0, 0))
_b_spec = pl.BlockSpec((1, D), lambda i: (0, 0))

_stage_first = pl.pallas_call(
    _stage_first_body,
    grid=(G,),
    in_specs=[_row_spec, _w_spec, _cnt_spec, _cnt_spec],
    out_specs=_row_spec,
    out_shape=jax.ShapeDtypeStruct((NPAD, D), jnp.float32),
)

_stage_mid = pl.pallas_call(
    _stage_mid_body,
    grid=(G,),
    in_specs=[_prt_spec, _prt_spec, _row_spec, _cnt_spec, _cnt_spec, _b_spec,
              _w_spec],
    out_specs=_row_spec,
    out_shape=jax.ShapeDtypeStruct((NPAD, D), jnp.float32),
)

_stage_last = pl.pallas_call(
    _stage_last_body,
    grid=(G,),
    in_specs=[_prt_spec, _prt_spec, _row_spec, _cnt_spec, _cnt_spec, _b_spec,
              pl.BlockSpec((D, 1), lambda i: (0, 0)),
              pl.BlockSpec((1, 1), lambda i: (0, 0))],
    out_specs=pl.BlockSpec((BM, 1), lambda i: (i, 0)),
    out_shape=jax.ShapeDtypeStruct((NPAD, 1), jnp.float32),
)


def kernel(x, edge_index, W1, b1, W2, b2, W3, b3, Wf, bf):
    x_pad = jnp.zeros((NPAD, D), jnp.float32).at[:N].set(x)
    pad = EPAD - E
    src_t = jnp.concatenate(
        [edge_index[0], jnp.full((pad,), N, jnp.int32)]).reshape(NW, CH, CB)
    dst_t = jnp.concatenate(
        [edge_index[1], jnp.full((pad,), N, jnp.int32)]).reshape(NW, CH, CB)

    cc = _sc_counts(dst_t)
    c0, c1 = cc[0:1], cc[1:2]
    b1r = b1.reshape(1, D)
    b2r = b2.reshape(1, D)
    b3r = b3.reshape(1, D)
    bfr = bf.reshape(1, 1)

    g = _stage_first(x_pad, W1, c0, c1)
    pp = _sc_scatter(g, src_t, dst_t)
    g = _stage_mid(pp[0:1], pp[1:2], g, c0, c1, b1r, W2)
    pp = _sc_scatter(g, src_t, dst_t)
    g = _stage_mid(pp[0:1], pp[1:2], g, c0, c1, b2r, W3)
    pp = _sc_scatter(g, src_t, dst_t)
    out = _stage_last(pp[0:1], pp[1:2], g, c0, c1, b3r, Wf, bfr)
    return out[:N]
